# Initial kernel scaffold; baseline (speedup 1.0000x reference)
import jax
import jax.numpy as jnp
from jax.experimental import pallas as pl


def kernel(x, edges, W0_1, b0_1, W1_1, b1_1, W0_2, b0_2, W1_2, b1_2, W0_3, b0_3, W1_3, b1_3, W0_4, b0_4, W1_4, b1_4, W0_5, b0_5, W1_5, b1_5):
    raise NotImplementedError("write your pallas kernel here")


# speedup vs baseline: 30.9904x; 30.9904x over previous
import functools

import jax
import jax.numpy as jnp
from jax import lax
from jax.experimental import pallas as pl
from jax.experimental.pallas import tpu as pltpu
from jax.experimental.pallas import tpu_sc as plsc

N = 100000
NP = 100096
E2 = 6400000
NC, NS = 2, 16
NW = NC * NS
EW = E2 // NW
C = 1000
G = EW // C
RPT = NP // NS
ZR = 368

BR = 4000
GRID = N // BR


@functools.partial(
    pl.kernel,
    out_type=jax.ShapeDtypeStruct((NC, NP, 16), jnp.float32),
    mesh=plsc.VectorSubcoreMesh(
        core_axis_name="c", subcore_axis_name="s",
        num_cores=NC, num_subcores=NS),
    scratch_types=[
        pltpu.VMEM((C,), jnp.int32),
        pltpu.VMEM((C,), jnp.int32),
        pltpu.VMEM((C, 16), jnp.float32),
        pltpu.VMEM_SHARED((NP, 16), jnp.float32),
        pltpu.SemaphoreType.DMA,
    ],
    compiler_params=pltpu.CompilerParams(use_tc_tiling_on_sc=False),
)
def _nbr_sum(tbl, srcr, dstr, outr, idx_s, idx_d, rows, acc, gsem):
    c = lax.axis_index("c")
    s = lax.axis_index("s")
    wid = c * NS + s

    def _z(i, carry):
        rows[i, :] = jnp.zeros((16,), jnp.float32)
        return carry
    lax.fori_loop(0, ZR, _z, 0)
    for r in range(RPT // ZR):
        pltpu.sync_copy(rows.at[pl.ds(0, ZR), :],
                        acc.at[pl.ds(s * RPT + r * ZR, ZR), :])
    plsc.subcore_barrier()

    base = wid * EW

    def _edges(g, carry):
        off = base + g * C
        pltpu.sync_copy(srcr.at[pl.ds(off, C)], idx_s)
        pltpu.sync_copy(dstr.at[pl.ds(off, C)], idx_d)
        pltpu.async_copy(tbl.at[idx_s], rows, gsem).wait()
        pltpu.sync_copy(rows, acc.at[idx_d], add=True)
        return carry
    lax.fori_loop(0, G, _edges, 0)

    plsc.subcore_barrier()
    pltpu.sync_copy(acc.at[pl.ds(s * RPT, RPT), :],
                    outr.at[c, pl.ds(s * RPT, RPT), :])


def _row_spec(w):
    return pl.BlockSpec((BR, w), lambda i: (i, 0))


def _pair_spec():
    return pl.BlockSpec((2, BR, 16), lambda i: (0, i, 0))


def _w_spec(a, b):
    return pl.BlockSpec((a, b), lambda i: (0, 0))


def _d1_body(t1, s1, w0, b0, w1, b1, v2o, dego):
    ssum = s1[0] + s1[1]
    ax = ssum[:, 0:1]
    deg = ssum[:, 1:2]
    xc = t1[...][:, 0:1]
    agg = ax * w1[...] + deg * b1[...]
    v2o[...] = jnp.maximum(xc * w0[...] + b0[...] + agg, 0.0)
    dego[...] = jnp.broadcast_to(deg, (BR, 16))


def _d2_body(v2, s2, dg, w0, b0, w1, b1, v3ao, v3bo):
    ssum = s2[0] + s2[1]
    deg = dg[...][:, 0:1]
    h = (jnp.dot(v2[...], w0[...], preferred_element_type=jnp.float32)
         + b0[...]
         + jnp.dot(ssum, w1[...], preferred_element_type=jnp.float32)
         + deg * b1[...])
    h = jnp.maximum(h, 0.0)
    v3ao[...] = h[:, :16]
    v3bo[...] = h[:, 16:]


def _d3_body(v3a, v3b, s3a, s3b, dg, w0, b0, w1, b1, w14, b14,
             v4o, h4ao, h4bo):
    v3 = jnp.concatenate([v3a[...], v3b[...]], axis=1)
    sa = s3a[0] + s3a[1]
    sb = s3b[0] + s3b[1]
    deg = dg[...][:, 0:1]
    agg = (jnp.dot(sa, w1[...][:16, :], preferred_element_type=jnp.float32)
           + jnp.dot(sb, w1[...][16:, :], preferred_element_type=jnp.float32)
           + deg * b1[...])
    v4 = jnp.maximum(
        jnp.dot(v3, w0[...], preferred_element_type=jnp.float32)
        + b0[...] + agg, 0.0)
    v4o[...] = v4
    h4 = jnp.dot(v4, w14[...], preferred_element_type=jnp.float32) + b14[...]
    h4ao[...] = h4[:, :16]
    h4bo[...] = h4[:, 16:]


def _d4_body(v4, s4a, s4b, w0, b0, w15, b15, v5o, h5o):
    agg = jnp.concatenate([s4a[0] + s4a[1], s4b[0] + s4b[1]], axis=1)
    v5 = jnp.maximum(
        jnp.dot(v4[...], w0[...], preferred_element_type=jnp.float32)
        + b0[...] + agg, 0.0)
    v5o[...] = v5
    h5 = jnp.dot(v5, w15[...], preferred_element_type=jnp.float32) + b15[...]
    col = lax.broadcasted_iota(jnp.int32, (1, 16), 1)
    h5o[...] = jnp.where(col == 0, h5, 0.0)


def _d5_body(v5, s5, w0, b0, oo):
    agg = (s5[0] + s5[1])[:, 0:1]
    o = (jnp.dot(v5[...], w0[...], preferred_element_type=jnp.float32)
         + b0[...] + agg)
    oo[...] = jax.nn.sigmoid(o)


def kernel(x, edges, W0_1, b0_1, W1_1, b1_1, W0_2, b0_2, W1_2, b1_2,
           W0_3, b0_3, W1_3, b1_3, W0_4, b0_4, W1_4, b1_4,
           W0_5, b0_5, W1_5, b1_5):
    f32 = jnp.float32
    src = jnp.concatenate([edges[:, 1], edges[:, 0]])
    dst = jnp.concatenate([edges[:, 0], edges[:, 1]])

    xT = x.reshape(N, 1)
    t1 = jnp.concatenate(
        [xT, jnp.ones((N, 1), f32), jnp.zeros((N, 14), f32)], axis=1)

    s1 = _nbr_sum(t1, src, dst)

    v2, deg16 = pl.pallas_call(
        _d1_body,
        grid=(GRID,),
        in_specs=[_row_spec(16), _pair_spec(), _w_spec(1, 16), _w_spec(1, 16),
                  _w_spec(1, 16), _w_spec(1, 16)],
        out_specs=[_row_spec(16), _row_spec(16)],
        out_shape=[jax.ShapeDtypeStruct((N, 16), f32),
                   jax.ShapeDtypeStruct((N, 16), f32)],
    )(t1, s1, W0_1, b0_1.reshape(1, 16), W1_1, b1_1.reshape(1, 16))

    s2 = _nbr_sum(v2, src, dst)

    v3a, v3b = pl.pallas_call(
        _d2_body,
        grid=(GRID,),
        in_specs=[_row_spec(16), _pair_spec(), _row_spec(16),
                  _w_spec(16, 32), _w_spec(1, 32), _w_spec(16, 32),
                  _w_spec(1, 32)],
        out_specs=[_row_spec(16), _row_spec(16)],
        out_shape=[jax.ShapeDtypeStruct((N, 16), f32),
                   jax.ShapeDtypeStruct((N, 16), f32)],
    )(v2, s2, deg16, W0_2, b0_2.reshape(1, 32), W1_2, b1_2.reshape(1, 32))

    s3a = _nbr_sum(v3a, src, dst)
    s3b = _nbr_sum(v3b, src, dst)

    v4, h4a, h4b = pl.pallas_call(
        _d3_body,
        grid=(GRID,),
        in_specs=[_row_spec(16), _row_spec(16), _pair_spec(), _pair_spec(),
                  _row_spec(16), _w_spec(32, 64), _w_spec(1, 64),
                  _w_spec(32, 64), _w_spec(1, 64), _w_spec(64, 32),
                  _w_spec(1, 32)],
        out_specs=[_row_spec(64), _row_spec(16), _row_spec(16)],
        out_shape=[jax.ShapeDtypeStruct((N, 64), f32),
                   jax.ShapeDtypeStruct((N, 16), f32),
                   jax.ShapeDtypeStruct((N, 16), f32)],
    )(v3a, v3b, s3a, s3b, deg16, W0_3, b0_3.reshape(1, 64), W1_3,
      b1_3.reshape(1, 64), W1_4, b1_4.reshape(1, 32))

    s4a = _nbr_sum(h4a, src, dst)
    s4b = _nbr_sum(h4b, src, dst)

    v5, h5p = pl.pallas_call(
        _d4_body,
        grid=(GRID,),
        in_specs=[_row_spec(64), _pair_spec(), _pair_spec(),
                  _w_spec(64, 32), _w_spec(1, 32), _w_spec(32, 1),
                  _w_spec(1, 1)],
        out_specs=[_row_spec(32), _row_spec(16)],
        out_shape=[jax.ShapeDtypeStruct((N, 32), f32),
                   jax.ShapeDtypeStruct((N, 16), f32)],
    )(v4, s4a, s4b, W0_4, b0_4.reshape(1, 32), W1_5, b1_5.reshape(1, 1))

    s5 = _nbr_sum(h5p, src, dst)

    o = pl.pallas_call(
        _d5_body,
        grid=(GRID,),
        in_specs=[_row_spec(32), _pair_spec(), _w_spec(32, 1), _w_spec(1, 1)],
        out_specs=[_row_spec(1)],
        out_shape=[jax.ShapeDtypeStruct((N, 1), f32)],
    )(v5, s5, W0_5, b0_5.reshape(1, 1))[0]

    return o.reshape(1, N)


# speedup vs baseline: 45.0997x; 1.4553x over previous
import functools

import jax
import jax.numpy as jnp
from jax import lax
from jax.experimental import pallas as pl
from jax.experimental.pallas import tpu as pltpu
from jax.experimental.pallas import tpu_sc as plsc

N = 100000
NP = 100096
E2 = 6400000
NC, NS = 2, 16
NW = NC * NS
EW = E2 // NW
C = 800
G = EW // C
RPT = NP // NS
ZR = 368

BR = 4000
GRID = N // BR


@functools.partial(
    pl.kernel,
    out_type=jax.ShapeDtypeStruct((NC, NP, 16), jnp.float32),
    mesh=plsc.VectorSubcoreMesh(
        core_axis_name="c", subcore_axis_name="s",
        num_cores=NC, num_subcores=NS),
    scratch_types=[
        pltpu.VMEM((C,), jnp.int32),
        pltpu.VMEM((C,), jnp.int32),
        pltpu.VMEM((C, 16), jnp.float32),
        pltpu.VMEM((C,), jnp.int32),
        pltpu.VMEM((C,), jnp.int32),
        pltpu.VMEM((C, 16), jnp.float32),
        pltpu.VMEM_SHARED((NP, 16), jnp.float32),
        pltpu.SemaphoreType.DMA,
        pltpu.SemaphoreType.DMA,
    ],
    compiler_params=pltpu.CompilerParams(use_tc_tiling_on_sc=False),
)
def _nbr_sum(tbl, srcr, dstr, outr, isa, ida, rowsa, isb, idb, rowsb,
             acc, sema, semb):
    c = lax.axis_index("c")
    s = lax.axis_index("s")
    wid = c * NS + s

    def _z(i, carry):
        rowsa[i, :] = jnp.zeros((16,), jnp.float32)
        return carry
    lax.fori_loop(0, ZR, _z, 0)
    for r in range(RPT // ZR):
        pltpu.sync_copy(rowsa.at[pl.ds(0, ZR), :],
                        acc.at[pl.ds(s * RPT + r * ZR, ZR), :])
    plsc.subcore_barrier()

    base = wid * EW

    def stage(g, isx, idx, rowsx, semx):
        off = base + g * C
        pltpu.sync_copy(srcr.at[pl.ds(off, C)], isx)
        pltpu.sync_copy(dstr.at[pl.ds(off, C)], idx)
        pltpu.async_copy(tbl.at[isx], rowsx, semx)

    def finish(isx, idx, rowsx, semx):
        pltpu.make_async_copy(tbl.at[isx], rowsx, semx).wait()
        pltpu.sync_copy(rowsx, acc.at[idx], add=True)

    K = G // 2
    stage(0, isa, ida, rowsa, sema)

    def _pair(k, carry):
        g = 2 * k
        stage(g + 1, isb, idb, rowsb, semb)
        finish(isa, ida, rowsa, sema)

        @pl.when(k + 1 < K)
        def _():
            stage(g + 2, isa, ida, rowsa, sema)
        finish(isb, idb, rowsb, semb)
        return carry
    lax.fori_loop(0, K, _pair, 0)

    plsc.subcore_barrier()
    pltpu.sync_copy(acc.at[pl.ds(s * RPT, RPT), :],
                    outr.at[c, pl.ds(s * RPT, RPT), :])


def _row_spec(w):
    return pl.BlockSpec((BR, w), lambda i: (i, 0))


def _pair_spec():
    return pl.BlockSpec((2, BR, 16), lambda i: (0, i, 0))


def _w_spec(a, b):
    return pl.BlockSpec((a, b), lambda i: (0, 0))


def _d1_body(t1, s1, w0, b0, w1, b1, v2o, dego):
    ssum = s1[0] + s1[1]
    ax = ssum[:, 0:1]
    deg = ssum[:, 1:2]
    xc = t1[...][:, 0:1]
    agg = ax * w1[...] + deg * b1[...]
    v2o[...] = jnp.maximum(xc * w0[...] + b0[...] + agg, 0.0)
    dego[...] = jnp.broadcast_to(deg, (BR, 16))


def _d2_body(v2, s2, dg, w0, b0, w1, b1, v3ao, v3bo):
    ssum = s2[0] + s2[1]
    deg = dg[...][:, 0:1]
    h = (jnp.dot(v2[...], w0[...], preferred_element_type=jnp.float32)
         + b0[...]
         + jnp.dot(ssum, w1[...], preferred_element_type=jnp.float32)
         + deg * b1[...])
    h = jnp.maximum(h, 0.0)
    v3ao[...] = h[:, :16]
    v3bo[...] = h[:, 16:]


def _d3_body(v3a, v3b, s3a, s3b, dg, w0, b0, w1, b1, w14, b14,
             v4o, h4ao, h4bo):
    v3 = jnp.concatenate([v3a[...], v3b[...]], axis=1)
    sa = s3a[0] + s3a[1]
    sb = s3b[0] + s3b[1]
    deg = dg[...][:, 0:1]
    agg = (jnp.dot(sa, w1[...][:16, :], preferred_element_type=jnp.float32)
           + jnp.dot(sb, w1[...][16:, :], preferred_element_type=jnp.float32)
           + deg * b1[...])
    v4 = jnp.maximum(
        jnp.dot(v3, w0[...], preferred_element_type=jnp.float32)
        + b0[...] + agg, 0.0)
    v4o[...] = v4
    h4 = jnp.dot(v4, w14[...], preferred_element_type=jnp.float32) + b14[...]
    h4ao[...] = h4[:, :16]
    h4bo[...] = h4[:, 16:]


def _d4_body(v4, s4a, s4b, w0, b0, w15, b15, v5o, h5o):
    agg = jnp.concatenate([s4a[0] + s4a[1], s4b[0] + s4b[1]], axis=1)
    v5 = jnp.maximum(
        jnp.dot(v4[...], w0[...], preferred_element_type=jnp.float32)
        + b0[...] + agg, 0.0)
    v5o[...] = v5
    h5 = jnp.dot(v5, w15[...], preferred_element_type=jnp.float32) + b15[...]
    col = lax.broadcasted_iota(jnp.int32, (1, 16), 1)
    h5o[...] = jnp.where(col == 0, h5, 0.0)


def _d5_body(v5, s5, w0, b0, oo):
    agg = (s5[0] + s5[1])[:, 0:1]
    o = (jnp.dot(v5[...], w0[...], preferred_element_type=jnp.float32)
         + b0[...] + agg)
    oo[...] = jax.nn.sigmoid(o)


def kernel(x, edges, W0_1, b0_1, W1_1, b1_1, W0_2, b0_2, W1_2, b1_2,
           W0_3, b0_3, W1_3, b1_3, W0_4, b0_4, W1_4, b1_4,
           W0_5, b0_5, W1_5, b1_5):
    f32 = jnp.float32
    src = jnp.concatenate([edges[:, 1], edges[:, 0]])
    dst = jnp.concatenate([edges[:, 0], edges[:, 1]])

    xT = x.reshape(N, 1)
    t1 = jnp.concatenate(
        [xT, jnp.ones((N, 1), f32), jnp.zeros((N, 14), f32)], axis=1)

    s1 = _nbr_sum(t1, src, dst)

    v2, deg16 = pl.pallas_call(
        _d1_body,
        grid=(GRID,),
        in_specs=[_row_spec(16), _pair_spec(), _w_spec(1, 16), _w_spec(1, 16),
                  _w_spec(1, 16), _w_spec(1, 16)],
        out_specs=[_row_spec(16), _row_spec(16)],
        out_shape=[jax.ShapeDtypeStruct((N, 16), f32),
                   jax.ShapeDtypeStruct((N, 16), f32)],
    )(t1, s1, W0_1, b0_1.reshape(1, 16), W1_1, b1_1.reshape(1, 16))

    s2 = _nbr_sum(v2, src, dst)

    v3a, v3b = pl.pallas_call(
        _d2_body,
        grid=(GRID,),
        in_specs=[_row_spec(16), _pair_spec(), _row_spec(16),
                  _w_spec(16, 32), _w_spec(1, 32), _w_spec(16, 32),
                  _w_spec(1, 32)],
        out_specs=[_row_spec(16), _row_spec(16)],
        out_shape=[jax.ShapeDtypeStruct((N, 16), f32),
                   jax.ShapeDtypeStruct((N, 16), f32)],
    )(v2, s2, deg16, W0_2, b0_2.reshape(1, 32), W1_2, b1_2.reshape(1, 32))

    s3a = _nbr_sum(v3a, src, dst)
    s3b = _nbr_sum(v3b, src, dst)

    v4, h4a, h4b = pl.pallas_call(
        _d3_body,
        grid=(GRID,),
        in_specs=[_row_spec(16), _row_spec(16), _pair_spec(), _pair_spec(),
                  _row_spec(16), _w_spec(32, 64), _w_spec(1, 64),
                  _w_spec(32, 64), _w_spec(1, 64), _w_spec(64, 32),
                  _w_spec(1, 32)],
        out_specs=[_row_spec(64), _row_spec(16), _row_spec(16)],
        out_shape=[jax.ShapeDtypeStruct((N, 64), f32),
                   jax.ShapeDtypeStruct((N, 16), f32),
                   jax.ShapeDtypeStruct((N, 16), f32)],
    )(v3a, v3b, s3a, s3b, deg16, W0_3, b0_3.reshape(1, 64), W1_3,
      b1_3.reshape(1, 64), W1_4, b1_4.reshape(1, 32))

    s4a = _nbr_sum(h4a, src, dst)
    s4b = _nbr_sum(h4b, src, dst)

    v5, h5p = pl.pallas_call(
        _d4_body,
        grid=(GRID,),
        in_specs=[_row_spec(64), _pair_spec(), _pair_spec(),
                  _w_spec(64, 32), _w_spec(1, 32), _w_spec(32, 1),
                  _w_spec(1, 1)],
        out_specs=[_row_spec(32), _row_spec(16)],
        out_shape=[jax.ShapeDtypeStruct((N, 32), f32),
                   jax.ShapeDtypeStruct((N, 16), f32)],
    )(v4, s4a, s4b, W0_4, b0_4.reshape(1, 32), W1_5, b1_5.reshape(1, 1))

    s5 = _nbr_sum(h5p, src, dst)

    o = pl.pallas_call(
        _d5_body,
        grid=(GRID,),
        in_specs=[_row_spec(32), _pair_spec(), _w_spec(32, 1), _w_spec(1, 1)],
        out_specs=[_row_spec(1)],
        out_shape=[jax.ShapeDtypeStruct((N, 1), f32)],
    )(v5, s5, W0_5, b0_5.reshape(1, 1))[0]

    return o.reshape(1, N)


# speedup vs baseline: 50.6434x; 1.1229x over previous
import functools

import jax
import jax.numpy as jnp
from jax import lax
from jax.experimental import pallas as pl
from jax.experimental.pallas import tpu as pltpu
from jax.experimental.pallas import tpu_sc as plsc

N = 100000
NP = 100096
E2 = 6400000
NC, NS = 2, 16
NW = NC * NS
EW = E2 // NW
C = 400
G = EW // C
RPT = NP // NS
ZR = 368

BR = 4000
GRID = N // BR


@functools.partial(
    pl.kernel,
    out_type=jax.ShapeDtypeStruct((NC, NP, 16), jnp.float32),
    mesh=plsc.VectorSubcoreMesh(
        core_axis_name="c", subcore_axis_name="s",
        num_cores=NC, num_subcores=NS),
    scratch_types=[
        pltpu.VMEM((C,), jnp.int32),
        pltpu.VMEM((C,), jnp.int32),
        pltpu.VMEM((C, 16), jnp.float32),
        pltpu.VMEM((C,), jnp.int32),
        pltpu.VMEM((C,), jnp.int32),
        pltpu.VMEM((C, 16), jnp.float32),
        pltpu.VMEM((C,), jnp.int32),
        pltpu.VMEM((C,), jnp.int32),
        pltpu.VMEM((C, 16), jnp.float32),
        pltpu.VMEM_SHARED((NP, 16), jnp.float32),
        pltpu.SemaphoreType.DMA,
        pltpu.SemaphoreType.DMA,
        pltpu.SemaphoreType.DMA,
        pltpu.SemaphoreType.DMA,
        pltpu.SemaphoreType.DMA,
        pltpu.SemaphoreType.DMA,
    ],
    compiler_params=pltpu.CompilerParams(use_tc_tiling_on_sc=False),
)
def _nbr_sum(tbl, srcr, dstr, outr,
             is0, id0, r0, is1, id1, r1, is2, id2, r2,
             acc, gs0, gs1, gs2, es0, es1, es2):
    c = lax.axis_index("c")
    s = lax.axis_index("s")
    wid = c * NS + s

    def _z(i, carry):
        r0[i, :] = jnp.zeros((16,), jnp.float32)
        return carry
    lax.fori_loop(0, ZR, _z, 0)
    for r in range(RPT // ZR):
        pltpu.sync_copy(r0.at[pl.ds(0, ZR), :],
                        acc.at[pl.ds(s * RPT + r * ZR, ZR), :])
    plsc.subcore_barrier()

    base = wid * EW
    bufs = [(is0, id0, r0, gs0, es0),
            (is1, id1, r1, gs1, es1),
            (is2, id2, r2, gs2, es2)]

    def idx_start(g, b):
        isx, idx, _, _, esx = b
        off = base + g * C
        pltpu.async_copy(srcr.at[pl.ds(off, C)], isx, esx)
        pltpu.async_copy(dstr.at[pl.ds(off, C)], idx, esx)

    def gather_start(g, b):
        isx, idx, rowsx, gsx, esx = b
        off = base + g * C
        pltpu.make_async_copy(srcr.at[pl.ds(off, C)], isx, esx).wait()
        pltpu.make_async_copy(dstr.at[pl.ds(off, C)], idx, esx).wait()
        pltpu.async_copy(tbl.at[isx], rowsx, gsx)

    def scat(b):
        isx, idx, rowsx, gsx, _ = b
        pltpu.make_async_copy(tbl.at[isx], rowsx, gsx).wait()
        pltpu.sync_copy(rowsx, acc.at[idx], add=True)

    idx_start(0, bufs[0])
    idx_start(1, bufs[1])
    idx_start(2, bufs[2])
    gather_start(0, bufs[0])
    gather_start(1, bufs[1])

    KK = (G + 2) // 3

    def _rot(k, carry):
        for j in range(3):
            g = 3 * k + j
            b = bufs[j]
            z = bufs[(j + 2) % 3]

            @pl.when(g < G)
            def _():
                scat(b)

            @pl.when(g + 3 < G)
            def _():
                idx_start(g + 3, b)

            @pl.when(g + 2 < G)
            def _():
                gather_start(g + 2, z)
        return carry
    lax.fori_loop(0, KK, _rot, 0)

    plsc.subcore_barrier()
    pltpu.sync_copy(acc.at[pl.ds(s * RPT, RPT), :],
                    outr.at[c, pl.ds(s * RPT, RPT), :])


def _row_spec(w):
    return pl.BlockSpec((BR, w), lambda i: (i, 0))


def _pair_spec():
    return pl.BlockSpec((2, BR, 16), lambda i: (0, i, 0))


def _w_spec(a, b):
    return pl.BlockSpec((a, b), lambda i: (0, 0))


def _d1_body(t1, s1, w0, b0, w1, b1, v2o, dego):
    ssum = s1[0] + s1[1]
    ax = ssum[:, 0:1]
    deg = ssum[:, 1:2]
    xc = t1[...][:, 0:1]
    agg = ax * w1[...] + deg * b1[...]
    v2o[...] = jnp.maximum(xc * w0[...] + b0[...] + agg, 0.0)
    dego[...] = jnp.broadcast_to(deg, (BR, 16))


def _d2_body(v2, s2, dg, w0, b0, w1, b1, v3ao, v3bo):
    ssum = s2[0] + s2[1]
    deg = dg[...][:, 0:1]
    h = (jnp.dot(v2[...], w0[...], preferred_element_type=jnp.float32)
         + b0[...]
         + jnp.dot(ssum, w1[...], preferred_element_type=jnp.float32)
         + deg * b1[...])
    h = jnp.maximum(h, 0.0)
    v3ao[...] = h[:, :16]
    v3bo[...] = h[:, 16:]


def _d3_body(v3a, v3b, s3a, s3b, dg, w0, b0, w1, b1, w14, b14,
             v4o, h4ao, h4bo):
    v3 = jnp.concatenate([v3a[...], v3b[...]], axis=1)
    sa = s3a[0] + s3a[1]
    sb = s3b[0] + s3b[1]
    deg = dg[...][:, 0:1]
    agg = (jnp.dot(sa, w1[...][:16, :], preferred_element_type=jnp.float32)
           + jnp.dot(sb, w1[...][16:, :], preferred_element_type=jnp.float32)
           + deg * b1[...])
    v4 = jnp.maximum(
        jnp.dot(v3, w0[...], preferred_element_type=jnp.float32)
        + b0[...] + agg, 0.0)
    v4o[...] = v4
    h4 = jnp.dot(v4, w14[...], preferred_element_type=jnp.float32) + b14[...]
    h4ao[...] = h4[:, :16]
    h4bo[...] = h4[:, 16:]


def _d4_body(v4, s4a, s4b, w0, b0, w15, b15, v5o, h5o):
    agg = jnp.concatenate([s4a[0] + s4a[1], s4b[0] + s4b[1]], axis=1)
    v5 = jnp.maximum(
        jnp.dot(v4[...], w0[...], preferred_element_type=jnp.float32)
        + b0[...] + agg, 0.0)
    v5o[...] = v5
    h5 = jnp.dot(v5, w15[...], preferred_element_type=jnp.float32) + b15[...]
    col = lax.broadcasted_iota(jnp.int32, (1, 16), 1)
    h5o[...] = jnp.where(col == 0, h5, 0.0)


def _d5_body(v5, s5, w0, b0, oo):
    agg = (s5[0] + s5[1])[:, 0:1]
    o = (jnp.dot(v5[...], w0[...], preferred_element_type=jnp.float32)
         + b0[...] + agg)
    oo[...] = jax.nn.sigmoid(o)


def kernel(x, edges, W0_1, b0_1, W1_1, b1_1, W0_2, b0_2, W1_2, b1_2,
           W0_3, b0_3, W1_3, b1_3, W0_4, b0_4, W1_4, b1_4,
           W0_5, b0_5, W1_5, b1_5):
    f32 = jnp.float32
    src = jnp.concatenate([edges[:, 1], edges[:, 0]])
    dst = jnp.concatenate([edges[:, 0], edges[:, 1]])

    xT = x.reshape(N, 1)
    t1 = jnp.concatenate(
        [xT, jnp.ones((N, 1), f32), jnp.zeros((N, 14), f32)], axis=1)

    s1 = _nbr_sum(t1, src, dst)

    v2, deg16 = pl.pallas_call(
        _d1_body,
        grid=(GRID,),
        in_specs=[_row_spec(16), _pair_spec(), _w_spec(1, 16), _w_spec(1, 16),
                  _w_spec(1, 16), _w_spec(1, 16)],
        out_specs=[_row_spec(16), _row_spec(16)],
        out_shape=[jax.ShapeDtypeStruct((N, 16), f32),
                   jax.ShapeDtypeStruct((N, 16), f32)],
    )(t1, s1, W0_1, b0_1.reshape(1, 16), W1_1, b1_1.reshape(1, 16))

    s2 = _nbr_sum(v2, src, dst)

    v3a, v3b = pl.pallas_call(
        _d2_body,
        grid=(GRID,),
        in_specs=[_row_spec(16), _pair_spec(), _row_spec(16),
                  _w_spec(16, 32), _w_spec(1, 32), _w_spec(16, 32),
                  _w_spec(1, 32)],
        out_specs=[_row_spec(16), _row_spec(16)],
        out_shape=[jax.ShapeDtypeStruct((N, 16), f32),
                   jax.ShapeDtypeStruct((N, 16), f32)],
    )(v2, s2, deg16, W0_2, b0_2.reshape(1, 32), W1_2, b1_2.reshape(1, 32))

    s3a = _nbr_sum(v3a, src, dst)
    s3b = _nbr_sum(v3b, src, dst)

    v4, h4a, h4b = pl.pallas_call(
        _d3_body,
        grid=(GRID,),
        in_specs=[_row_spec(16), _row_spec(16), _pair_spec(), _pair_spec(),
                  _row_spec(16), _w_spec(32, 64), _w_spec(1, 64),
                  _w_spec(32, 64), _w_spec(1, 64), _w_spec(64, 32),
                  _w_spec(1, 32)],
        out_specs=[_row_spec(64), _row_spec(16), _row_spec(16)],
        out_shape=[jax.ShapeDtypeStruct((N, 64), f32),
                   jax.ShapeDtypeStruct((N, 16), f32),
                   jax.ShapeDtypeStruct((N, 16), f32)],
    )(v3a, v3b, s3a, s3b, deg16, W0_3, b0_3.reshape(1, 64), W1_3,
      b1_3.reshape(1, 64), W1_4, b1_4.reshape(1, 32))

    s4a = _nbr_sum(h4a, src, dst)
    s4b = _nbr_sum(h4b, src, dst)

    v5, h5p = pl.pallas_call(
        _d4_body,
        grid=(GRID,),
        in_specs=[_row_spec(64), _pair_spec(), _pair_spec(),
                  _w_spec(64, 32), _w_spec(1, 32), _w_spec(32, 1),
                  _w_spec(1, 1)],
        out_specs=[_row_spec(32), _row_spec(16)],
        out_shape=[jax.ShapeDtypeStruct((N, 32), f32),
                   jax.ShapeDtypeStruct((N, 16), f32)],
    )(v4, s4a, s4b, W0_4, b0_4.reshape(1, 32), W1_5, b1_5.reshape(1, 1))

    s5 = _nbr_sum(h5p, src, dst)

    o = pl.pallas_call(
        _d5_body,
        grid=(GRID,),
        in_specs=[_row_spec(32), _pair_spec(), _w_spec(32, 1), _w_spec(1, 1)],
        out_specs=[_row_spec(1)],
        out_shape=[jax.ShapeDtypeStruct((N, 1), f32)],
    )(v5, s5, W0_5, b0_5.reshape(1, 1))[0]

    return o.reshape(1, N)


# speedup vs baseline: 60.5766x; 1.1961x over previous
import functools

import jax
import jax.numpy as jnp
from jax import lax
from jax.experimental import pallas as pl
from jax.experimental.pallas import tpu as pltpu
from jax.experimental.pallas import tpu_sc as plsc

N = 100000
NP = 100096
E2 = 6400000
NC, NS = 2, 16
NW = NC * NS
EW = E2 // NW
C = 400
G = EW // C
RPT = NP // NS
ZR = 368

BR = 4000
GRID = N // BR


@functools.partial(
    pl.kernel,
    out_type=jax.ShapeDtypeStruct((NC, NP, 16), jnp.float32),
    mesh=plsc.VectorSubcoreMesh(
        core_axis_name="c", subcore_axis_name="s",
        num_cores=NC, num_subcores=NS),
    scratch_types=[
        pltpu.VMEM((C,), jnp.int32),
        pltpu.VMEM((C,), jnp.int32),
        pltpu.VMEM((C, 16), jnp.float32),
        pltpu.VMEM((C,), jnp.int32),
        pltpu.VMEM((C,), jnp.int32),
        pltpu.VMEM((C, 16), jnp.float32),
        pltpu.VMEM((C,), jnp.int32),
        pltpu.VMEM((C,), jnp.int32),
        pltpu.VMEM((C, 16), jnp.float32),
        pltpu.VMEM_SHARED((NP, 16), jnp.float32),
        pltpu.SemaphoreType.DMA,
        pltpu.SemaphoreType.DMA,
        pltpu.SemaphoreType.DMA,
        pltpu.SemaphoreType.DMA,
        pltpu.SemaphoreType.DMA,
        pltpu.SemaphoreType.DMA,
    ],
    compiler_params=pltpu.CompilerParams(use_tc_tiling_on_sc=False),
)
def _nbr_sum(tbl, srcr, dstr, outr,
             is0, id0, r0, is1, id1, r1, is2, id2, r2,
             acc, gs0, gs1, gs2, es0, es1, es2):
    c = lax.axis_index("c")
    s = lax.axis_index("s")
    wid = c * NS + s

    def _z(i, carry):
        r0[i, :] = jnp.zeros((16,), jnp.float32)
        return carry
    lax.fori_loop(0, ZR, _z, 0)
    for r in range(RPT // ZR):
        pltpu.sync_copy(r0.at[pl.ds(0, ZR), :],
                        acc.at[pl.ds(s * RPT + r * ZR, ZR), :])
    plsc.subcore_barrier()

    base = wid * EW
    bufs = [(is0, id0, r0, gs0, es0),
            (is1, id1, r1, gs1, es1),
            (is2, id2, r2, gs2, es2)]

    def idx_start(g, b):
        isx, idx, _, _, esx = b
        off = base + g * C
        pltpu.async_copy(srcr.at[pl.ds(off, C)], isx, esx)
        pltpu.async_copy(dstr.at[pl.ds(off, C)], idx, esx)

    def gather_start(g, b):
        isx, idx, rowsx, gsx, esx = b
        off = base + g * C
        pltpu.make_async_copy(srcr.at[pl.ds(off, C)], isx, esx).wait()
        pltpu.make_async_copy(dstr.at[pl.ds(off, C)], idx, esx).wait()
        pltpu.async_copy(tbl.at[isx], rowsx, gsx)

    def scat(b):
        isx, idx, rowsx, gsx, _ = b
        pltpu.make_async_copy(tbl.at[isx], rowsx, gsx).wait()
        pltpu.sync_copy(rowsx, acc.at[idx], add=True)

    idx_start(0, bufs[0])
    idx_start(1, bufs[1])
    idx_start(2, bufs[2])
    gather_start(0, bufs[0])
    gather_start(1, bufs[1])

    KK = (G + 2) // 3

    def _rot(k, carry):
        for j in range(3):
            g = 3 * k + j
            b = bufs[j]
            z = bufs[(j + 2) % 3]

            @pl.when(g < G)
            def _():
                scat(b)

            @pl.when(g + 3 < G)
            def _():
                idx_start(g + 3, b)

            @pl.when(g + 2 < G)
            def _():
                gather_start(g + 2, z)
        return carry
    lax.fori_loop(0, KK, _rot, 0)

    plsc.subcore_barrier()
    pltpu.sync_copy(acc.at[pl.ds(s * RPT, RPT), :],
                    outr.at[c, pl.ds(s * RPT, RPT), :])


NP8 = NP // 8
BR8 = 736
GRID8 = NP8 // BR8


def _p_spec(w8):
    return pl.BlockSpec((BR8, w8), lambda i: (i, 0))


def _pp_spec():
    return pl.BlockSpec((2, BR8, 128), lambda i: (0, i, 0))


def _w_spec(a, b):
    return pl.BlockSpec((a, b), lambda i: (0, 0))


def _dot(a, b):
    return jnp.dot(a, b, preferred_element_type=jnp.float32)


def _d1_body(t1, s1, w0, b0, w1, b1, v2o, dego):
    ss = (s1[0] + s1[1]).reshape(BR8, 8, 16)
    t3 = t1[...].reshape(BR8, 8, 16)
    ax = ss[:, :, 0:1]
    deg = ss[:, :, 1:2]
    xc = t3[:, :, 0:1]
    w0r = w0[...].reshape(1, 1, 16)
    b0r = b0[...].reshape(1, 1, 16)
    w1r = w1[...].reshape(1, 1, 16)
    b1r = b1[...].reshape(1, 1, 16)
    agg = ax * w1r + deg * b1r
    v2 = jnp.maximum(xc * w0r + b0r + agg, 0.0)
    v2o[...] = v2.reshape(BR8, 128)
    dego[...] = jnp.broadcast_to(deg, (BR8, 8, 16)).reshape(BR8, 128)


def _d2_body(v2, s2, dg, w0, b0, w1, b1, v3ao, v3bo):
    ss = s2[0] + s2[1]
    deg = dg[...].reshape(BR8, 8, 16)[:, :, 0:1]
    b1r = b1[...].reshape(1, 1, 32)
    h = (_dot(v2[...], w0[...]) + b0[...] + _dot(ss, w1[...])
         + (deg * b1r).reshape(BR8, 256))
    h3 = jnp.maximum(h, 0.0).reshape(BR8, 8, 32)
    v3ao[...] = h3[:, :, :16].reshape(BR8, 128)
    v3bo[...] = h3[:, :, 16:].reshape(BR8, 128)


def _d3_body(v3a, v3b, s3a, s3b, dg, w0a, w0b, b0, w1a, w1b, b1, w14, b14,
             v4o, h4ao, h4bo):
    sa = s3a[0] + s3a[1]
    sb = s3b[0] + s3b[1]
    deg = dg[...].reshape(BR8, 8, 16)[:, :, 0:1]
    b1r = b1[...].reshape(1, 1, 64)
    agg = (_dot(sa, w1a[...]) + _dot(sb, w1b[...])
           + (deg * b1r).reshape(BR8, 512))
    v4 = jnp.maximum(
        _dot(v3a[...], w0a[...]) + _dot(v3b[...], w0b[...]) + b0[...] + agg,
        0.0)
    v4o[...] = v4
    h4 = (_dot(v4, w14[...]) + b14[...]).reshape(BR8, 8, 32)
    h4ao[...] = h4[:, :, :16].reshape(BR8, 128)
    h4bo[...] = h4[:, :, 16:].reshape(BR8, 128)


def _d4_body(v4, s4a, s4b, w0, b0, w15, b15, v5o, h5o):
    sa = (s4a[0] + s4a[1]).reshape(BR8, 8, 16)
    sb = (s4b[0] + s4b[1]).reshape(BR8, 8, 16)
    agg = jnp.concatenate([sa, sb], axis=2).reshape(BR8, 256)
    v5 = jnp.maximum(_dot(v4[...], w0[...]) + b0[...] + agg, 0.0)
    v5o[...] = v5
    h5 = (_dot(v5, w15[...]) + b15[...]).reshape(BR8, 8, 1)
    col = lax.broadcasted_iota(jnp.int32, (1, 1, 16), 2)
    h5o[...] = jnp.where(col == 0, h5, 0.0).reshape(BR8, 128)


def _d5_body(v5, s5, w0, b0, oo):
    agg = (s5[0] + s5[1]).reshape(BR8, 8, 16)[:, :, 0:1].reshape(BR8, 8)
    o = _dot(v5[...], w0[...]) + b0[...] + agg
    oo[...] = jax.nn.sigmoid(o)


def kernel(x, edges, W0_1, b0_1, W1_1, b1_1, W0_2, b0_2, W1_2, b1_2,
           W0_3, b0_3, W1_3, b1_3, W0_4, b0_4, W1_4, b1_4,
           W0_5, b0_5, W1_5, b1_5):
    f32 = jnp.float32
    src = jnp.concatenate([edges[:, 1], edges[:, 0]])
    dst = jnp.concatenate([edges[:, 0], edges[:, 1]])

    eye8 = jnp.eye(8, dtype=f32)

    def bd(w):
        return jnp.kron(eye8, w)

    def bt(b):
        return jnp.tile(b, 8).reshape(1, -1)

    xp = jnp.pad(x.reshape(N), (0, NP - N)).reshape(NP8, 8, 1)
    t1p3 = jnp.concatenate(
        [xp, jnp.ones((NP8, 8, 1), f32), jnp.zeros((NP8, 8, 14), f32)],
        axis=2)
    t1p = t1p3.reshape(NP8, 128)

    s1 = _nbr_sum(t1p.reshape(NP, 16), src, dst)

    v2p, degp = pl.pallas_call(
        _d1_body,
        grid=(GRID8,),
        in_specs=[_p_spec(128), _pp_spec(), _w_spec(1, 16), _w_spec(1, 16),
                  _w_spec(1, 16), _w_spec(1, 16)],
        out_specs=[_p_spec(128), _p_spec(128)],
        out_shape=[jax.ShapeDtypeStruct((NP8, 128), f32),
                   jax.ShapeDtypeStruct((NP8, 128), f32)],
    )(t1p, s1.reshape(2, NP8, 128), W0_1, b0_1.reshape(1, 16), W1_1,
      b1_1.reshape(1, 16))

    s2 = _nbr_sum(v2p.reshape(NP, 16), src, dst)

    v3ap, v3bp = pl.pallas_call(
        _d2_body,
        grid=(GRID8,),
        in_specs=[_p_spec(128), _pp_spec(), _p_spec(128),
                  _w_spec(128, 256), _w_spec(1, 256), _w_spec(128, 256),
                  _w_spec(1, 32)],
        out_specs=[_p_spec(128), _p_spec(128)],
        out_shape=[jax.ShapeDtypeStruct((NP8, 128), f32),
                   jax.ShapeDtypeStruct((NP8, 128), f32)],
    )(v2p, s2.reshape(2, NP8, 128), degp, bd(W0_2), bt(b0_2), bd(W1_2),
      b1_2.reshape(1, 32))

    s3a = _nbr_sum(v3ap.reshape(NP, 16), src, dst)
    s3b = _nbr_sum(v3bp.reshape(NP, 16), src, dst)

    v4p, h4ap, h4bp = pl.pallas_call(
        _d3_body,
        grid=(GRID8,),
        in_specs=[_p_spec(128), _p_spec(128), _pp_spec(), _pp_spec(),
                  _p_spec(128), _w_spec(128, 512), _w_spec(128, 512),
                  _w_spec(1, 512), _w_spec(128, 512), _w_spec(128, 512),
                  _w_spec(1, 64), _w_spec(512, 256), _w_spec(1, 256)],
        out_specs=[_p_spec(512), _p_spec(128), _p_spec(128)],
        out_shape=[jax.ShapeDtypeStruct((NP8, 512), f32),
                   jax.ShapeDtypeStruct((NP8, 128), f32),
                   jax.ShapeDtypeStruct((NP8, 128), f32)],
    )(v3ap, v3bp, s3a.reshape(2, NP8, 128), s3b.reshape(2, NP8, 128), degp,
      bd(W0_3[:16, :]), bd(W0_3[16:, :]), bt(b0_3),
      bd(W1_3[:16, :]), bd(W1_3[16:, :]), b1_3.reshape(1, 64),
      bd(W1_4), bt(b1_4))

    s4a = _nbr_sum(h4ap.reshape(NP, 16), src, dst)
    s4b = _nbr_sum(h4bp.reshape(NP, 16), src, dst)

    v5p, h5p = pl.pallas_call(
        _d4_body,
        grid=(GRID8,),
        in_specs=[_p_spec(512), _pp_spec(), _pp_spec(),
                  _w_spec(512, 256), _w_spec(1, 256), _w_spec(256, 8),
                  _w_spec(1, 8)],
        out_specs=[_p_spec(256), _p_spec(128)],
        out_shape=[jax.ShapeDtypeStruct((NP8, 256), f32),
                   jax.ShapeDtypeStruct((NP8, 128), f32)],
    )(v4p, s4a.reshape(2, NP8, 128), s4b.reshape(2, NP8, 128),
      bd(W0_4), bt(b0_4), bd(W1_5), bt(b1_5))

    s5 = _nbr_sum(h5p.reshape(NP, 16), src, dst)

    o = pl.pallas_call(
        _d5_body,
        grid=(GRID8,),
        in_specs=[_p_spec(256), _pp_spec(), _w_spec(256, 8), _w_spec(1, 8)],
        out_specs=[_p_spec(8)],
        out_shape=[jax.ShapeDtypeStruct((NP8, 8), f32)],
    )(v5p, s5.reshape(2, NP8, 128), bd(W0_5), bt(b0_5))[0]

    return o.reshape(NP)[:N].reshape(1, N)


# speedup vs baseline: 66.6138x; 1.0997x over previous
import functools

import jax
import jax.numpy as jnp
from jax import lax
from jax.experimental import pallas as pl
from jax.experimental.pallas import tpu as pltpu
from jax.experimental.pallas import tpu_sc as plsc

N = 100000
NP = 100096
E2 = 6400000
NC, NS = 2, 16
NW = NC * NS
EW = E2 // NW
C = 400
G = EW // C
RPT = NP // NS
ZR = 368

BR = 4000
GRID = N // BR


@functools.partial(
    pl.kernel,
    out_type=jax.ShapeDtypeStruct((NC, NP, 16), jnp.float32),
    mesh=plsc.VectorSubcoreMesh(
        core_axis_name="c", subcore_axis_name="s",
        num_cores=NC, num_subcores=NS),
    scratch_types=[
        pltpu.VMEM((C,), jnp.int32),
        pltpu.VMEM((C,), jnp.int32),
        pltpu.VMEM((C, 16), jnp.float32),
        pltpu.VMEM((C,), jnp.int32),
        pltpu.VMEM((C,), jnp.int32),
        pltpu.VMEM((C, 16), jnp.float32),
        pltpu.VMEM((C,), jnp.int32),
        pltpu.VMEM((C,), jnp.int32),
        pltpu.VMEM((C, 16), jnp.float32),
        pltpu.VMEM_SHARED((NP, 16), jnp.float32),
        pltpu.SemaphoreType.DMA,
        pltpu.SemaphoreType.DMA,
        pltpu.SemaphoreType.DMA,
        pltpu.SemaphoreType.DMA,
        pltpu.SemaphoreType.DMA,
        pltpu.SemaphoreType.DMA,
    ],
    compiler_params=pltpu.CompilerParams(use_tc_tiling_on_sc=False),
)
def _nbr_sum(tbl, srcr, dstr, outr,
             is0, id0, r0, is1, id1, r1, is2, id2, r2,
             acc, gs0, gs1, gs2, es0, es1, es2):
    c = lax.axis_index("c")
    s = lax.axis_index("s")
    wid = c * NS + s

    def _z(i, carry):
        r0[i, :] = jnp.zeros((16,), jnp.float32)
        return carry
    lax.fori_loop(0, ZR, _z, 0)
    for r in range(RPT // ZR):
        pltpu.sync_copy(r0.at[pl.ds(0, ZR), :],
                        acc.at[pl.ds(s * RPT + r * ZR, ZR), :])
    plsc.subcore_barrier()

    base = wid * EW
    bufs = [(is0, id0, r0, gs0, es0),
            (is1, id1, r1, gs1, es1),
            (is2, id2, r2, gs2, es2)]

    def idx_start(g, b):
        isx, idx, _, _, esx = b
        off = base + g * C
        pltpu.async_copy(srcr.at[pl.ds(off, C)], isx, esx)
        pltpu.async_copy(dstr.at[pl.ds(off, C)], idx, esx)

    def gather_start(g, b):
        isx, idx, rowsx, gsx, esx = b
        off = base + g * C
        pltpu.make_async_copy(srcr.at[pl.ds(off, C)], isx, esx).wait()
        pltpu.make_async_copy(dstr.at[pl.ds(off, C)], idx, esx).wait()
        pltpu.async_copy(tbl.at[isx], rowsx, gsx)

    def scat(b):
        isx, idx, rowsx, gsx, _ = b
        pltpu.make_async_copy(tbl.at[isx], rowsx, gsx).wait()
        pltpu.sync_copy(rowsx, acc.at[idx], add=True)

    idx_start(0, bufs[0])
    idx_start(1, bufs[1])
    idx_start(2, bufs[2])
    gather_start(0, bufs[0])
    gather_start(1, bufs[1])

    KK = (G + 2) // 3

    def _rot(k, carry):
        for j in range(3):
            g = 3 * k + j
            b = bufs[j]
            z = bufs[(j + 2) % 3]

            @pl.when(g < G)
            def _():
                scat(b)

            @pl.when(g + 3 < G)
            def _():
                idx_start(g + 3, b)

            @pl.when(g + 2 < G)
            def _():
                gather_start(g + 2, z)
        return carry
    lax.fori_loop(0, KK, _rot, 0)

    plsc.subcore_barrier()
    pltpu.sync_copy(acc.at[pl.ds(s * RPT, RPT), :],
                    outr.at[c, pl.ds(s * RPT, RPT), :])


NP8 = NP // 8
BR8 = 3128
GRID8 = NP8 // BR8


def _p_spec(w8):
    return pl.BlockSpec((BR8, w8), lambda i: (i, 0))


def _pp_spec():
    return pl.BlockSpec((2, BR8, 128), lambda i: (0, i, 0))


def _w_spec(a, b):
    return pl.BlockSpec((a, b), lambda i: (0, 0))


def _dot(a, b):
    return jnp.dot(a, b, preferred_element_type=jnp.float32)


def _d1_body(t1, s1, k0, kf, kd, v2o, dego):
    ss = s1[0] + s1[1]
    v2o[...] = jnp.maximum(_dot(t1[...], k0[...]) + _dot(ss, kf[...]), 0.0)
    dego[...] = _dot(ss, kd[...])


def _d2_body(v2, s2, dg, w0, b0, w1, db, pa, pb, v3ao, v3bo):
    ss = s2[0] + s2[1]
    h = (_dot(v2[...], w0[...]) + b0[...] + _dot(ss, w1[...])
         + _dot(dg[...], db[...]))
    h = jnp.maximum(h, 0.0)
    v3ao[...] = _dot(h, pa[...])
    v3bo[...] = _dot(h, pb[...])


def _d3_body(v3a, v3b, s3a, s3b, dg, w0a, w0b, b0, w1a, w1b, db, w14, b14,
             pa, pb, v4o, h4ao, h4bo):
    sa = s3a[0] + s3a[1]
    sb = s3b[0] + s3b[1]
    agg = _dot(sa, w1a[...]) + _dot(sb, w1b[...]) + _dot(dg[...], db[...])
    v4 = jnp.maximum(
        _dot(v3a[...], w0a[...]) + _dot(v3b[...], w0b[...]) + b0[...] + agg,
        0.0)
    v4o[...] = v4
    h4 = _dot(v4, w14[...]) + b14[...]
    h4ao[...] = _dot(h4, pa[...])
    h4bo[...] = _dot(h4, pb[...])


def _d4_body(v4, s4a, s4b, w0, b0, ea, eb, w15, b15, v5o, h5o):
    agg = _dot(s4a[0] + s4a[1], ea[...]) + _dot(s4b[0] + s4b[1], eb[...])
    v5 = jnp.maximum(_dot(v4[...], w0[...]) + b0[...] + agg, 0.0)
    v5o[...] = v5
    h5o[...] = _dot(v5, w15[...]) + b15[...]


def _d5_body(v5, s5, w0, b0, q, oo):
    agg = _dot(s5[0] + s5[1], q[...])
    oo[...] = jax.nn.sigmoid(_dot(v5[...], w0[...]) + b0[...] + agg)


def kernel(x, edges, W0_1, b0_1, W1_1, b1_1, W0_2, b0_2, W1_2, b1_2,
           W0_3, b0_3, W1_3, b1_3, W0_4, b0_4, W1_4, b1_4,
           W0_5, b0_5, W1_5, b1_5):
    f32 = jnp.float32
    src = jnp.concatenate([edges[:, 1], edges[:, 0]])
    dst = jnp.concatenate([edges[:, 0], edges[:, 1]])

    eye8 = jnp.eye(8, dtype=f32)
    eye16 = jnp.eye(16, dtype=f32)

    def bd(w):
        return jnp.kron(eye8, w)

    def bt(b):
        return jnp.tile(b, 8).reshape(1, -1)

    z16 = jnp.zeros((16, 16), f32)
    k0 = bd(z16.at[0].set(W0_1[0]).at[1].set(b0_1))
    kf = bd(z16.at[0].set(W1_1[0]).at[1].set(b1_1))
    kd = bd(z16.at[1].set(jnp.ones((16,), f32)))
    db2 = bd(jnp.zeros((16, 32), f32).at[0].set(b1_2))
    db3 = bd(jnp.zeros((16, 64), f32).at[0].set(b1_3))
    pa = bd(jnp.eye(32, 16, dtype=f32))
    pb = bd(jnp.zeros((32, 16), f32).at[16:, :].set(eye16))
    ea = bd(jnp.eye(16, 32, dtype=f32))
    eb = bd(jnp.zeros((16, 32), f32).at[:, 16:].set(eye16))
    w15p = bd(jnp.zeros((32, 16), f32).at[:, 0].set(W1_5[:, 0]))
    b15p = bt(jnp.zeros((16,), f32).at[0].set(b1_5[0]))
    q5 = bd(jnp.zeros((16, 1), f32).at[0, 0].set(1.0))

    xp = jnp.pad(x.reshape(N), (0, NP - N)).reshape(NP8, 8, 1)
    t1p = jnp.concatenate(
        [xp, jnp.ones((NP8, 8, 1), f32), jnp.zeros((NP8, 8, 14), f32)],
        axis=2).reshape(NP8, 128)

    s1 = _nbr_sum(t1p.reshape(NP, 16), src, dst)

    v2p, degp = pl.pallas_call(
        _d1_body,
        grid=(GRID8,),
        in_specs=[_p_spec(128), _pp_spec(), _w_spec(128, 128),
                  _w_spec(128, 128), _w_spec(128, 128)],
        out_specs=[_p_spec(128), _p_spec(128)],
        out_shape=[jax.ShapeDtypeStruct((NP8, 128), f32),
                   jax.ShapeDtypeStruct((NP8, 128), f32)],
    )(t1p, s1.reshape(2, NP8, 128), k0, kf, kd)

    s2 = _nbr_sum(v2p.reshape(NP, 16), src, dst)

    v3ap, v3bp = pl.pallas_call(
        _d2_body,
        grid=(GRID8,),
        in_specs=[_p_spec(128), _pp_spec(), _p_spec(128),
                  _w_spec(128, 256), _w_spec(1, 256), _w_spec(128, 256),
                  _w_spec(128, 256), _w_spec(256, 128), _w_spec(256, 128)],
        out_specs=[_p_spec(128), _p_spec(128)],
        out_shape=[jax.ShapeDtypeStruct((NP8, 128), f32),
                   jax.ShapeDtypeStruct((NP8, 128), f32)],
    )(v2p, s2.reshape(2, NP8, 128), degp, bd(W0_2), bt(b0_2), bd(W1_2),
      db2, pa, pb)

    s3a = _nbr_sum(v3ap.reshape(NP, 16), src, dst)
    s3b = _nbr_sum(v3bp.reshape(NP, 16), src, dst)

    v4p, h4ap, h4bp = pl.pallas_call(
        _d3_body,
        grid=(GRID8,),
        in_specs=[_p_spec(128), _p_spec(128), _pp_spec(), _pp_spec(),
                  _p_spec(128), _w_spec(128, 512), _w_spec(128, 512),
                  _w_spec(1, 512), _w_spec(128, 512), _w_spec(128, 512),
                  _w_spec(128, 512), _w_spec(512, 256), _w_spec(1, 256),
                  _w_spec(256, 128), _w_spec(256, 128)],
        out_specs=[_p_spec(512), _p_spec(128), _p_spec(128)],
        out_shape=[jax.ShapeDtypeStruct((NP8, 512), f32),
                   jax.ShapeDtypeStruct((NP8, 128), f32),
                   jax.ShapeDtypeStruct((NP8, 128), f32)],
    )(v3ap, v3bp, s3a.reshape(2, NP8, 128), s3b.reshape(2, NP8, 128), degp,
      bd(W0_3[:16, :]), bd(W0_3[16:, :]), bt(b0_3),
      bd(W1_3[:16, :]), bd(W1_3[16:, :]), db3,
      bd(W1_4), bt(b1_4), pa, pb)

    s4a = _nbr_sum(h4ap.reshape(NP, 16), src, dst)
    s4b = _nbr_sum(h4bp.reshape(NP, 16), src, dst)

    v5p, h5p = pl.pallas_call(
        _d4_body,
        grid=(GRID8,),
        in_specs=[_p_spec(512), _pp_spec(), _pp_spec(),
                  _w_spec(512, 256), _w_spec(1, 256), _w_spec(128, 256),
                  _w_spec(128, 256), _w_spec(256, 128), _w_spec(1, 128)],
        out_specs=[_p_spec(256), _p_spec(128)],
        out_shape=[jax.ShapeDtypeStruct((NP8, 256), f32),
                   jax.ShapeDtypeStruct((NP8, 128), f32)],
    )(v4p, s4a.reshape(2, NP8, 128), s4b.reshape(2, NP8, 128),
      bd(W0_4), bt(b0_4), ea, eb, w15p, b15p)

    s5 = _nbr_sum(h5p.reshape(NP, 16), src, dst)

    o = pl.pallas_call(
        _d5_body,
        grid=(GRID8,),
        in_specs=[_p_spec(256), _pp_spec(), _w_spec(256, 8), _w_spec(1, 8),
                  _w_spec(128, 8)],
        out_specs=[_p_spec(8)],
        out_shape=[jax.ShapeDtypeStruct((NP8, 8), f32)],
    )(v5p, s5.reshape(2, NP8, 128), bd(W0_5), bt(b0_5), q5)[0]

    return o.reshape(NP)[:N].reshape(1, N)
